# baseline (speedup 1.0000x reference)
import functools

import jax
import jax.numpy as jnp
from jax import lax
from jax.experimental import pallas as pl
from jax.experimental.pallas import tpu as pltpu
from jax.experimental.pallas import tpu_sc as plsc

N_NODES = 100000
G_SEG = 64
N_PAD = 100352
NR = 784
NPW = N_PAD // 16
NCORE = 2
NSUB = 16
NW = NCORE * NSUB
CHUNK = 128
INNER = 8


def _mesh():
    return plsc.VectorSubcoreMesh(
        core_axis_name="c", subcore_axis_name="s",
        num_cores=NCORE, num_subcores=NSUB)


def _zero_vmem_1d(ref, n):
    def body(i, _):
        ref[pl.ds(i * 16, 16)] = jnp.zeros((16,), jnp.float32)
        return 0
    lax.fori_loop(0, n // 16, body, 0)


def _sc_deg_body(rows_per_group, dst_hbm, out_hbm, dstb0, dstb1, acc,
                 lsem0, lsem1):
    c = lax.axis_index("c")
    s = lax.axis_index("s")
    w = c * NSUB + s
    g = w // 2
    h = w % 2
    lo = h * (N_PAD // 2)

    _zero_vmem_1d(acc, N_PAD // 2)

    dstb = [dstb0, dstb1]
    lsem = [lsem0, lsem1]
    n_outer = rows_per_group // INNER
    row0 = g * rows_per_group
    ones = jnp.ones((16,), jnp.float32)

    def issue(p, it):
        r0 = row0 + it * INNER
        pltpu.async_copy(dst_hbm.at[pl.ds(r0, INNER)], dstb[p], lsem[p])

    def drain(p):
        pltpu.make_async_copy(dst_hbm.at[pl.ds(0, INNER)], dstb[p],
                              lsem[p]).wait()

    issue(0, 0)

    def outer(kk, _):
        for p in range(2):
            it = 2 * kk + p
            drain(p)

            @pl.when(it + 1 < n_outer)
            def _():
                issue(1 - p, it + 1)

            for j in range(INNER):
                for q in range(CHUNK // 16):
                    dv = dstb[p][j, pl.ds(q * 16, 16)]
                    li = dv - lo
                    m = li.astype(jnp.uint32) < jnp.uint32(N_PAD // 2)
                    lis = jnp.where(m, li, 0)
                    plsc.addupdate_scatter(acc, [lis], ones, mask=m)
        return 0
    lax.fori_loop(0, n_outer // 2, outer, 0)

    pltpu.sync_copy(acc, out_hbm.at[g, h])


def _sc_deg(dst2d):
    rows = dst2d.shape[0]
    rpg = rows // (NW // 2)
    body = functools.partial(_sc_deg_body, rpg)
    return pl.kernel(
        body,
        out_type=jax.ShapeDtypeStruct((NW // 2, 2, N_PAD // 2), jnp.float32),
        mesh=_mesh(),
        scratch_types=[
            pltpu.VMEM((INNER, CHUNK), jnp.int32),
            pltpu.VMEM((INNER, CHUNK), jnp.int32),
            pltpu.VMEM((N_PAD // 2,), jnp.float32),
            pltpu.SemaphoreType.DMA,
            pltpu.SemaphoreType.DMA,
        ],
        compiler_params=pltpu.CompilerParams(use_tc_tiling_on_sc=False,
                                             needs_layout_passes=False),
    )(dst2d)


HALF = N_PAD // 2
NGRP = NW // 4


def _sc_spmv_body(rows_per_group, src_hbm, dst_hbm, vp_hbm, out_hbm,
                  srcb0, srcb1, dstb0, dstb1, vp_v, acc, lsem0, lsem1):
    c = lax.axis_index("c")
    s = lax.axis_index("s")
    w = c * NSUB + s
    g = w // 4
    r = w % 4
    f = r // 2
    h = r % 2
    lo = h * HALF

    _zero_vmem_1d(acc, HALF)
    pltpu.sync_copy(vp_hbm.at[f], vp_v)

    srcb = [srcb0, srcb1]
    dstb = [dstb0, dstb1]
    lsem = [lsem0, lsem1]
    n_outer = rows_per_group // INNER
    row0 = g * rows_per_group

    def issue(p, it):
        r0 = row0 + it * INNER
        pltpu.async_copy(src_hbm.at[pl.ds(r0, INNER)], srcb[p], lsem[p])
        pltpu.async_copy(dst_hbm.at[pl.ds(r0, INNER)], dstb[p], lsem[p])

    def drain(p):
        pltpu.make_async_copy(src_hbm.at[pl.ds(0, INNER)], srcb[p],
                              lsem[p]).wait()
        pltpu.make_async_copy(dst_hbm.at[pl.ds(0, INNER)], dstb[p],
                              lsem[p]).wait()

    issue(0, 0)

    def outer(kk, _):
        for p in range(2):
            it = 2 * kk + p
            drain(p)

            @pl.when(it + 1 < n_outer)
            def _():
                issue(1 - p, it + 1)

            for j in range(INNER):
                for q in range(CHUNK // 16):
                    sv = srcb[p][j, pl.ds(q * 16, 16)]
                    dv = dstb[p][j, pl.ds(q * 16, 16)]
                    cs = sv < HALF
                    pi = jnp.where(cs, sv, sv - HALF)
                    w32 = plsc.load_gather(vp_v, [pi])
                    vhi = plsc.bitcast(
                        jnp.bitwise_and(w32, jnp.int32(-65536)), jnp.float32)
                    vlo = plsc.bitcast(
                        jnp.left_shift(w32, 16), jnp.float32)
                    v = jnp.where(cs, vlo, vhi)
                    li = dv - lo
                    m = li.astype(jnp.uint32) < jnp.uint32(HALF)
                    lis = jnp.where(m, li, 0)
                    plsc.addupdate_scatter(acc, [lis], v, mask=m)
        return 0
    lax.fori_loop(0, n_outer // 2, outer, 0)

    pltpu.sync_copy(acc, out_hbm.at[g, f, h])


def _sc_spmv(src2d, dst2d, vp):
    rows = src2d.shape[0]
    rpg = rows // NGRP
    body = functools.partial(_sc_spmv_body, rpg)
    return pl.kernel(
        body,
        out_type=jax.ShapeDtypeStruct((NGRP, 2, 2, HALF), jnp.float32),
        mesh=_mesh(),
        scratch_types=[
            pltpu.VMEM((INNER, CHUNK), jnp.int32),
            pltpu.VMEM((INNER, CHUNK), jnp.int32),
            pltpu.VMEM((INNER, CHUNK), jnp.int32),
            pltpu.VMEM((INNER, CHUNK), jnp.int32),
            pltpu.VMEM((HALF,), jnp.int32),
            pltpu.VMEM((HALF,), jnp.float32),
            pltpu.SemaphoreType.DMA,
            pltpu.SemaphoreType.DMA,
        ],
        compiler_params=pltpu.CompilerParams(use_tc_tiling_on_sc=False,
                                             needs_layout_passes=False),
    )(src2d, dst2d, vp)


def _pack_pair(v):
    lob = lax.bitcast_convert_type(
        v[0:NR // 2, :].astype(jnp.bfloat16), jnp.uint16).astype(jnp.uint32)
    hib = lax.bitcast_convert_type(
        v[NR // 2:NR, :].astype(jnp.bfloat16), jnp.uint16).astype(jnp.uint32)
    return lax.bitcast_convert_type(
        jnp.bitwise_or(lob, jnp.left_shift(hib, 16)), jnp.int32)


def _tc_prep_body(dp, x0, x1, dis_o, inv_o, vp0_o, vp1_o):
    deg = dp[0] + 1.0
    for gi in range(1, NW // 2):
        deg = deg + dp[gi]
    dis = lax.rsqrt(deg)
    inv = 1.0 / deg
    dis_o[...] = dis
    inv_o[...] = inv
    vp0_o[...] = _pack_pair(x0[...] * dis)
    vp1_o[...] = _pack_pair(x1[...] * dis)


def _tc_prep(dp, x0, x1):
    sds = jax.ShapeDtypeStruct((NR, 128), jnp.float32)
    ids = jax.ShapeDtypeStruct((NR // 2, 128), jnp.int32)
    return pl.pallas_call(
        _tc_prep_body,
        out_shape=[sds, sds, ids, ids],
    )(dp, x0, x1)


def _tc_mid_body(ap, x0, x1, dis, inv, W1, b1, W2,
                 vt0_o, vt1_o, tf0_o, tf1_o):
    disv = dis[...]
    invv = inv[...]
    s0 = ap[0, 0]
    s1 = ap[0, 1]
    for gi in range(1, NGRP):
        s0 = s0 + ap[gi, 0]
        s1 = s1 + ap[gi, 1]
    y0 = disv * s0 + x0[...] * invv
    y1 = disv * s1 + x1[...] * invv
    t0 = jnp.zeros_like(y0)
    t1 = jnp.zeros_like(y0)
    for j in range(64):
        h = jnp.maximum(y0 * W1[0, j] + y1 * W1[1, j] + b1[j], 0.0)
        t0 = t0 + h * W2[j, 0]
        t1 = t1 + h * W2[j, 1]
    vt0_o[...] = _pack_pair(t0 * disv)
    vt1_o[...] = _pack_pair(t1 * disv)
    tf0_o[...] = t0 * invv
    tf1_o[...] = t1 * invv


def _tc_mid(ap, x0, x1, dis, inv, W1, b1, W2):
    sds = jax.ShapeDtypeStruct((NR, 128), jnp.float32)
    ids = jax.ShapeDtypeStruct((NR // 2, 128), jnp.int32)
    vspec = pl.BlockSpec(memory_space=pltpu.VMEM)
    sspec = pl.BlockSpec(memory_space=pltpu.SMEM)
    return pl.pallas_call(
        _tc_mid_body,
        out_shape=[ids, ids, sds, sds],
        in_specs=[vspec] * 5 + [sspec, sspec, sspec],
        out_specs=[vspec] * 4,
    )(ap, x0, x1, dis, inv, W1, b1, W2)


def _tc_final_body(bp, dis, tf0, tf1, batch_r, b2, out):
    disv = dis[...]
    s0 = bp[0, 0]
    s1 = bp[0, 1]
    for gi in range(1, NGRP):
        s0 = s0 + bp[gi, 0]
        s1 = s1 + bp[gi, 1]
    y0 = disv * s0 + tf0[...] + b2[0]
    y1 = disv * s1 + tf1[...] + b2[1]
    node = (lax.broadcasted_iota(jnp.int32, (NR, 128), 0) * 128
            + lax.broadcasted_iota(jnp.int32, (NR, 128), 1))
    valid = node < N_NODES
    neg = jnp.float32(-jnp.inf)
    bt = batch_r[...]
    p0 = []
    p1 = []
    for g in range(G_SEG):
        m = jnp.logical_and(bt == g, valid)
        p0.append(jnp.max(jnp.where(m, y0, neg)))
        p1.append(jnp.max(jnp.where(m, y1, neg)))
    pa = jnp.stack(p0)
    pb = jnp.stack(p1)
    mx = jnp.maximum(pa, pb)
    lse = mx + jnp.log(jnp.exp(pa - mx) + jnp.exp(pb - mx))
    out[0, :] = pa - lse
    out[1, :] = pb - lse


def _tc_final(bp, dis, tf0, tf1, batch_r, b2):
    vspec = pl.BlockSpec(memory_space=pltpu.VMEM)
    sspec = pl.BlockSpec(memory_space=pltpu.SMEM)
    return pl.pallas_call(
        _tc_final_body,
        out_shape=jax.ShapeDtypeStruct((2, G_SEG), jnp.float32),
        in_specs=[vspec] * 5 + [sspec],
        out_specs=vspec,
    )(bp, dis, tf0, tf1, batch_r, b2)


def _soa(v):
    return jnp.pad(v, (0, N_PAD - v.shape[0])).reshape(NR, 128)


def kernel(x, ei, batch, W1, b1, W2, b2):
    E = ei.shape[1]
    rpw = -(-E // (NW * CHUNK * 2 * INNER)) * 2 * INNER
    rows = rpw * NW
    e_pad = rows * CHUNK

    src = jnp.concatenate(
        [ei[0], jnp.full((e_pad - E,), N_NODES, jnp.int32)]).reshape(rows, CHUNK)
    dst = jnp.concatenate(
        [ei[1], jnp.full((e_pad - E,), N_NODES, jnp.int32)]).reshape(rows, CHUNK)

    x0 = _soa(x[:, 0])
    x1 = _soa(x[:, 1])

    degp = _sc_deg(dst)
    dp = degp.reshape(NW // 2, NR, 128)

    dis, inv, vp0, vp1 = _tc_prep(dp, x0, x1)

    vp_x = jnp.stack([vp0.reshape(-1), vp1.reshape(-1)])
    acc1 = _sc_spmv(src, dst, vp_x)
    ap = acc1.reshape(NGRP, 2, NR, 128)

    vt0, vt1, tf0, tf1 = _tc_mid(ap, x0, x1, dis, inv, W1, b1, W2)

    vp_t = jnp.stack([vt0.reshape(-1), vt1.reshape(-1)])
    acc2 = _sc_spmv(src, dst, vp_t)
    bp = acc2.reshape(NGRP, 2, NR, 128)

    batch_r = jnp.pad(batch, (0, N_PAD - batch.shape[0]),
                      constant_values=G_SEG - 1).reshape(NR, 128)

    out = _tc_final(bp, dis, tf0, tf1, batch_r, b2)
    return out.T


# speedup vs baseline: 1.0830x; 1.0830x over previous
import functools

import jax
import jax.numpy as jnp
from jax import lax
from jax.experimental import pallas as pl
from jax.experimental.pallas import tpu as pltpu
from jax.experimental.pallas import tpu_sc as plsc

N_NODES = 100000
G_SEG = 64
N_PAD = 100352
NR = 784
NPW = N_PAD // 16
NCORE = 2
NSUB = 16
NW = NCORE * NSUB
CHUNK = 128
INNER = 8


def _mesh():
    return plsc.VectorSubcoreMesh(
        core_axis_name="c", subcore_axis_name="s",
        num_cores=NCORE, num_subcores=NSUB)


def _zero_vmem_1d(ref, n):
    def body(i, _):
        ref[pl.ds(i * 16, 16)] = jnp.zeros((16,), jnp.float32)
        return 0
    lax.fori_loop(0, n // 16, body, 0)


def _sc_deg_body(rows_per_worker, dst_hbm, out_hbm, dstbuf, dstbuf2, ones_v,
                 zbuf, obuf, deg_sp, ssem0, ssem1):
    c = lax.axis_index("c")
    s = lax.axis_index("s")
    w = c * NSUB + s

    _zero_vmem_1d(zbuf, NPW)

    def fill_ones(i, _):
        ones_v[pl.ds(i * 16, 16)] = jnp.ones((16,), jnp.float32)
        return 0
    lax.fori_loop(0, CHUNK // 16, fill_ones, 0)

    pltpu.sync_copy(zbuf, deg_sp.at[pl.ds(s * NPW, NPW)])
    plsc.subcore_barrier()

    n_outer = rows_per_worker // INNER
    row0 = w * rows_per_worker
    dstb = [dstbuf, dstbuf2]
    ssem = [ssem0, ssem1]

    def outer(k, _):
        for p in range(2):
            it = 2 * k + p

            @pl.when(k > 0)
            def _drain():
                for j in range(INNER):
                    pltpu.make_async_copy(
                        ones_v, deg_sp.at[dstb[p].at[j]], ssem[p]).wait()

            r0 = row0 + it * INNER
            pltpu.sync_copy(dst_hbm.at[pl.ds(r0, INNER)], dstb[p])
            for j in range(INNER):
                pltpu.async_copy(
                    ones_v, deg_sp.at[dstb[p].at[j]], ssem[p], add=True)
        return 0
    lax.fori_loop(0, n_outer // 2, outer, 0)
    for p in range(2):
        for j in range(INNER):
            pltpu.make_async_copy(
                ones_v, deg_sp.at[dstb[p].at[j]], ssem[p]).wait()

    plsc.subcore_barrier()
    pltpu.sync_copy(deg_sp.at[pl.ds(s * NPW, NPW)], obuf)
    pltpu.sync_copy(obuf, out_hbm.at[c, pl.ds(s * NPW, NPW)])


def _sc_deg(dst2d):
    rows = dst2d.shape[0]
    rpw = rows // NW
    body = functools.partial(_sc_deg_body, rpw)
    return pl.kernel(
        body,
        out_type=jax.ShapeDtypeStruct((NCORE, N_PAD), jnp.float32),
        mesh=_mesh(),
        scratch_types=[
            pltpu.VMEM((INNER, CHUNK), jnp.int32),
            pltpu.VMEM((INNER, CHUNK), jnp.int32),
            pltpu.VMEM((CHUNK,), jnp.float32),
            pltpu.VMEM((NPW,), jnp.float32),
            pltpu.VMEM((NPW,), jnp.float32),
            pltpu.VMEM_SHARED((N_PAD,), jnp.float32),
            pltpu.SemaphoreType.DMA,
            pltpu.SemaphoreType.DMA,
        ],
        compiler_params=pltpu.CompilerParams(use_tc_tiling_on_sc=False),
    )(dst2d)


HALF = N_PAD // 2
NGRP = NW // 4


def _sc_spmv_body(rows_per_group, src_hbm, dst_hbm, vp_hbm, out_hbm,
                  srcb0, srcb1, dstb0, dstb1, vp_v, acc, lsem0, lsem1):
    c = lax.axis_index("c")
    s = lax.axis_index("s")
    w = c * NSUB + s
    g = w // 4
    r = w % 4
    f = r // 2
    h = r % 2
    lo = h * HALF

    _zero_vmem_1d(acc, HALF)
    pltpu.sync_copy(vp_hbm.at[f], vp_v)

    srcb = [srcb0, srcb1]
    dstb = [dstb0, dstb1]
    lsem = [lsem0, lsem1]
    n_outer = rows_per_group // INNER
    row0 = g * rows_per_group

    def issue(p, it):
        r0 = row0 + it * INNER
        pltpu.async_copy(src_hbm.at[pl.ds(r0, INNER)], srcb[p], lsem[p])
        pltpu.async_copy(dst_hbm.at[pl.ds(r0, INNER)], dstb[p], lsem[p])

    def drain(p):
        pltpu.make_async_copy(src_hbm.at[pl.ds(0, INNER)], srcb[p],
                              lsem[p]).wait()
        pltpu.make_async_copy(dst_hbm.at[pl.ds(0, INNER)], dstb[p],
                              lsem[p]).wait()

    issue(0, 0)

    def outer(kk, _):
        for p in range(2):
            it = 2 * kk + p
            drain(p)

            @pl.when(it + 1 < n_outer)
            def _():
                issue(1 - p, it + 1)

            for j in range(INNER):
                for q in range(CHUNK // 16):
                    sv = srcb[p][j, pl.ds(q * 16, 16)]
                    dv = dstb[p][j, pl.ds(q * 16, 16)]
                    cs = sv < HALF
                    pi = jnp.where(cs, sv, sv - HALF)
                    w32 = plsc.load_gather(vp_v, [pi])
                    vhi = plsc.bitcast(
                        jnp.bitwise_and(w32, jnp.int32(-65536)), jnp.float32)
                    vlo = plsc.bitcast(
                        jnp.left_shift(w32, 16), jnp.float32)
                    v = jnp.where(cs, vlo, vhi)
                    li = dv - lo
                    m = li.astype(jnp.uint32) < jnp.uint32(HALF)
                    lis = jnp.where(m, li, 0)
                    plsc.addupdate_scatter(acc, [lis], v, mask=m)
        return 0
    lax.fori_loop(0, n_outer // 2, outer, 0)

    pltpu.sync_copy(acc, out_hbm.at[g, f, h])


def _sc_spmv(src2d, dst2d, vp):
    rows = src2d.shape[0]
    rpg = rows // NGRP
    body = functools.partial(_sc_spmv_body, rpg)
    return pl.kernel(
        body,
        out_type=jax.ShapeDtypeStruct((NGRP, 2, 2, HALF), jnp.float32),
        mesh=_mesh(),
        scratch_types=[
            pltpu.VMEM((INNER, CHUNK), jnp.int32),
            pltpu.VMEM((INNER, CHUNK), jnp.int32),
            pltpu.VMEM((INNER, CHUNK), jnp.int32),
            pltpu.VMEM((INNER, CHUNK), jnp.int32),
            pltpu.VMEM((HALF,), jnp.int32),
            pltpu.VMEM((HALF,), jnp.float32),
            pltpu.SemaphoreType.DMA,
            pltpu.SemaphoreType.DMA,
        ],
        compiler_params=pltpu.CompilerParams(use_tc_tiling_on_sc=False,
                                             needs_layout_passes=False),
    )(src2d, dst2d, vp)


def _pack_pair(v):
    lob = lax.bitcast_convert_type(
        v[0:NR // 2, :].astype(jnp.bfloat16), jnp.uint16).astype(jnp.uint32)
    hib = lax.bitcast_convert_type(
        v[NR // 2:NR, :].astype(jnp.bfloat16), jnp.uint16).astype(jnp.uint32)
    return lax.bitcast_convert_type(
        jnp.bitwise_or(lob, jnp.left_shift(hib, 16)), jnp.int32)


def _tc_prep_body(d0, d1, x0, x1, dis_o, inv_o, vp0_o, vp1_o):
    deg = d0[...] + d1[...] + 1.0
    dis = lax.rsqrt(deg)
    inv = 1.0 / deg
    dis_o[...] = dis
    inv_o[...] = inv
    vp0_o[...] = _pack_pair(x0[...] * dis)
    vp1_o[...] = _pack_pair(x1[...] * dis)


def _tc_prep(d0, d1, x0, x1):
    sds = jax.ShapeDtypeStruct((NR, 128), jnp.float32)
    ids = jax.ShapeDtypeStruct((NR // 2, 128), jnp.int32)
    return pl.pallas_call(
        _tc_prep_body,
        out_shape=[sds, sds, ids, ids],
    )(d0, d1, x0, x1)


def _tc_mid_body(ap, x0, x1, dis, inv, W1, b1, W2,
                 vt0_o, vt1_o, tf0_o, tf1_o):
    disv = dis[...]
    invv = inv[...]
    s0 = ap[0, 0]
    s1 = ap[0, 1]
    for gi in range(1, NGRP):
        s0 = s0 + ap[gi, 0]
        s1 = s1 + ap[gi, 1]
    y0 = disv * s0 + x0[...] * invv
    y1 = disv * s1 + x1[...] * invv
    t0 = jnp.zeros_like(y0)
    t1 = jnp.zeros_like(y0)
    for j in range(64):
        h = jnp.maximum(y0 * W1[0, j] + y1 * W1[1, j] + b1[j], 0.0)
        t0 = t0 + h * W2[j, 0]
        t1 = t1 + h * W2[j, 1]
    vt0_o[...] = _pack_pair(t0 * disv)
    vt1_o[...] = _pack_pair(t1 * disv)
    tf0_o[...] = t0 * invv
    tf1_o[...] = t1 * invv


def _tc_mid(ap, x0, x1, dis, inv, W1, b1, W2):
    sds = jax.ShapeDtypeStruct((NR, 128), jnp.float32)
    ids = jax.ShapeDtypeStruct((NR // 2, 128), jnp.int32)
    vspec = pl.BlockSpec(memory_space=pltpu.VMEM)
    sspec = pl.BlockSpec(memory_space=pltpu.SMEM)
    return pl.pallas_call(
        _tc_mid_body,
        out_shape=[ids, ids, sds, sds],
        in_specs=[vspec] * 5 + [sspec, sspec, sspec],
        out_specs=[vspec] * 4,
    )(ap, x0, x1, dis, inv, W1, b1, W2)


def _tc_final_body(bp, dis, tf0, tf1, batch_r, b2, out):
    disv = dis[...]
    s0 = bp[0, 0]
    s1 = bp[0, 1]
    for gi in range(1, NGRP):
        s0 = s0 + bp[gi, 0]
        s1 = s1 + bp[gi, 1]
    y0 = disv * s0 + tf0[...] + b2[0]
    y1 = disv * s1 + tf1[...] + b2[1]
    node = (lax.broadcasted_iota(jnp.int32, (NR, 128), 0) * 128
            + lax.broadcasted_iota(jnp.int32, (NR, 128), 1))
    valid = node < N_NODES
    neg = jnp.float32(-jnp.inf)
    bt = batch_r[...]
    p0 = []
    p1 = []
    for g in range(G_SEG):
        m = jnp.logical_and(bt == g, valid)
        p0.append(jnp.max(jnp.where(m, y0, neg)))
        p1.append(jnp.max(jnp.where(m, y1, neg)))
    pa = jnp.stack(p0)
    pb = jnp.stack(p1)
    mx = jnp.maximum(pa, pb)
    lse = mx + jnp.log(jnp.exp(pa - mx) + jnp.exp(pb - mx))
    out[0, :] = pa - lse
    out[1, :] = pb - lse


def _tc_final(bp, dis, tf0, tf1, batch_r, b2):
    vspec = pl.BlockSpec(memory_space=pltpu.VMEM)
    sspec = pl.BlockSpec(memory_space=pltpu.SMEM)
    return pl.pallas_call(
        _tc_final_body,
        out_shape=jax.ShapeDtypeStruct((2, G_SEG), jnp.float32),
        in_specs=[vspec] * 5 + [sspec],
        out_specs=vspec,
    )(bp, dis, tf0, tf1, batch_r, b2)


def _soa(v):
    return jnp.pad(v, (0, N_PAD - v.shape[0])).reshape(NR, 128)


def kernel(x, ei, batch, W1, b1, W2, b2):
    E = ei.shape[1]
    rpw = -(-E // (NW * CHUNK * 2 * INNER)) * 2 * INNER
    rows = rpw * NW
    e_pad = rows * CHUNK

    src = jnp.concatenate(
        [ei[0], jnp.full((e_pad - E,), N_NODES, jnp.int32)]).reshape(rows, CHUNK)
    dst = jnp.concatenate(
        [ei[1], jnp.full((e_pad - E,), N_NODES, jnp.int32)]).reshape(rows, CHUNK)

    x0 = _soa(x[:, 0])
    x1 = _soa(x[:, 1])

    degp = _sc_deg(dst)
    d0 = degp[0].reshape(NR, 128)
    d1 = degp[1].reshape(NR, 128)

    dis, inv, vp0, vp1 = _tc_prep(d0, d1, x0, x1)

    vp_x = jnp.stack([vp0.reshape(-1), vp1.reshape(-1)])
    acc1 = _sc_spmv(src, dst, vp_x)
    ap = acc1.reshape(NGRP, 2, NR, 128)

    vt0, vt1, tf0, tf1 = _tc_mid(ap, x0, x1, dis, inv, W1, b1, W2)

    vp_t = jnp.stack([vt0.reshape(-1), vt1.reshape(-1)])
    acc2 = _sc_spmv(src, dst, vp_t)
    bp = acc2.reshape(NGRP, 2, NR, 128)

    batch_r = jnp.pad(batch, (0, N_PAD - batch.shape[0]),
                      constant_values=G_SEG - 1).reshape(NR, 128)

    out = _tc_final(bp, dis, tf0, tf1, batch_r, b2)
    return out.T
